# baseline (speedup 1.0000x reference)
import jax
import jax.numpy as jnp
from jax.experimental import pallas as pl

_TM = 1024


def _mm_filter_kernel(x_ref, w_ref, b_ref, o_ref):
    acc = jax.lax.dot_general(
        x_ref[...].astype(jnp.bfloat16), w_ref[...].astype(jnp.bfloat16),
        dimension_numbers=(((1,), (1,)), ((), ())),
        preferred_element_type=jnp.float32,
    )
    out = acc + b_ref[...]
    row_ok = jnp.all(jnp.isfinite(out), axis=1, keepdims=True)
    o_ref[...] = jnp.where(row_ok, out, jnp.zeros_like(out))


def kernel(x, W, b):
    M, K = x.shape
    N = W.shape[0]
    b2 = b.reshape(1, N)
    return pl.pallas_call(
        _mm_filter_kernel,
        grid=(M // _TM,),
        in_specs=[
            pl.BlockSpec((_TM, K), lambda i: (i, 0)),
            pl.BlockSpec((N, K), lambda i: (0, 0)),
            pl.BlockSpec((1, N), lambda i: (0, 0)),
        ],
        out_specs=pl.BlockSpec((_TM, N), lambda i: (i, 0)),
        out_shape=jax.ShapeDtypeStruct((M, N), jnp.float32),
    )(x, W, b2)


# speedup vs baseline: 1.0211x; 1.0211x over previous
import jax
import jax.numpy as jnp
from jax.experimental import pallas as pl
from jax.experimental.pallas import tpu as pltpu

_TM = 512


def _mm_filter_kernel(x_ref, w_ref, b_ref, o_ref, wb_ref):
    @pl.when(pl.program_id(0) == 0)
    def _pack_weights():
        wb_ref[...] = w_ref[...].astype(jnp.bfloat16)

    acc = jax.lax.dot_general(
        x_ref[...].astype(jnp.bfloat16), wb_ref[...],
        dimension_numbers=(((1,), (1,)), ((), ())),
        preferred_element_type=jnp.float32,
    )
    out = acc + b_ref[...]
    bad = jnp.sum(out * 0.0, axis=1, keepdims=True)
    o_ref[...] = jnp.where(bad == 0.0, out, jnp.zeros_like(out))


def kernel(x, W, b):
    M, K = x.shape
    N = W.shape[0]
    b2 = b.reshape(1, N)
    return pl.pallas_call(
        _mm_filter_kernel,
        grid=(M // _TM,),
        in_specs=[
            pl.BlockSpec((_TM, K), lambda i: (i, 0)),
            pl.BlockSpec((N, K), lambda i: (0, 0)),
            pl.BlockSpec((1, N), lambda i: (0, 0)),
        ],
        out_specs=pl.BlockSpec((_TM, N), lambda i: (i, 0)),
        out_shape=jax.ShapeDtypeStruct((M, N), jnp.float32),
        scratch_shapes=[pltpu.VMEM((N, K), jnp.bfloat16)],
    )(x, W, b2)


# speedup vs baseline: 1.0286x; 1.0073x over previous
import jax
import jax.numpy as jnp
from jax.experimental import pallas as pl
from jax.experimental.pallas import tpu as pltpu

_TM = 512


def _mm_filter_kernel(x_ref, w_ref, b_ref, o_ref, wb_ref):
    @pl.when(pl.program_id(0) == 0)
    def _pack_weights():
        wb_ref[...] = w_ref[...].astype(jnp.bfloat16)

    acc = jax.lax.dot_general(
        x_ref[...].astype(jnp.bfloat16), wb_ref[...],
        dimension_numbers=(((1,), (1,)), ((), ())),
        preferred_element_type=jnp.float32,
    )
    out = acc + b_ref[...]
    o_ref[...] = out
    bad = jnp.sum(out * 0.0, axis=1, keepdims=True)

    @pl.when(jnp.logical_not(jnp.all(bad == 0.0)))
    def _fix_rows():
        o_ref[...] = jnp.where(bad == 0.0, o_ref[...], jnp.zeros_like(out))


def kernel(x, W, b):
    M, K = x.shape
    N = W.shape[0]
    b2 = b.reshape(1, N)
    return pl.pallas_call(
        _mm_filter_kernel,
        grid=(M // _TM,),
        in_specs=[
            pl.BlockSpec((_TM, K), lambda i: (i, 0)),
            pl.BlockSpec((N, K), lambda i: (0, 0)),
            pl.BlockSpec((1, N), lambda i: (0, 0)),
        ],
        out_specs=pl.BlockSpec((_TM, N), lambda i: (i, 0)),
        out_shape=jax.ShapeDtypeStruct((M, N), jnp.float32),
        scratch_shapes=[pltpu.VMEM((N, K), jnp.bfloat16)],
    )(x, W, b2)


# speedup vs baseline: 1.0598x; 1.0304x over previous
import jax
import jax.numpy as jnp
from jax.experimental import pallas as pl
from jax.experimental.pallas import tpu as pltpu

_TM = 1024


def _mm_filter_kernel(x_ref, w_ref, b_ref, o_ref):
    acc = jax.lax.dot_general(
        x_ref[...], w_ref[...],
        dimension_numbers=(((1,), (1,)), ((), ())),
        preferred_element_type=jnp.float32,
    )
    out = acc + b_ref[...]
    o_ref[...] = out
    bad = jnp.sum(out * 0.0, axis=1, keepdims=True)

    @pl.when(jnp.logical_not(jnp.all(bad == 0.0)))
    def _fix_rows():
        o_ref[...] = jnp.where(bad == 0.0, o_ref[...], jnp.zeros_like(out))


def kernel(x, W, b):
    M, K = x.shape
    N = W.shape[0]
    b2 = b.reshape(1, N)
    return pl.pallas_call(
        _mm_filter_kernel,
        grid=(M // _TM,),
        in_specs=[
            pl.BlockSpec((_TM, K), lambda i: (i, 0)),
            pl.BlockSpec((N, K), lambda i: (0, 0)),
            pl.BlockSpec((1, N), lambda i: (0, 0)),
        ],
        out_specs=pl.BlockSpec((_TM, N), lambda i: (i, 0)),
        out_shape=jax.ShapeDtypeStruct((M, N), jnp.float32),
        compiler_params=pltpu.CompilerParams(
            dimension_semantics=("parallel",),
        ),
    )(x, W, b2)


# speedup vs baseline: 1.0606x; 1.0007x over previous
import jax
import jax.numpy as jnp
from jax.experimental import pallas as pl
from jax.experimental.pallas import tpu as pltpu

_TM = 1024
_NH = 1024


def _mm_filter_kernel(x_ref, w_ref, b_ref, o_ref):
    x = x_ref[...]
    h0 = jax.lax.dot_general(
        x, w_ref[0:_NH, :],
        dimension_numbers=(((1,), (1,)), ((), ())),
        preferred_element_type=jnp.float32,
    ) + b_ref[:, 0:_NH]
    o_ref[:, 0:_NH] = h0
    bad0 = jnp.sum(h0 * 0.0, axis=1, keepdims=True)

    h1 = jax.lax.dot_general(
        x, w_ref[_NH:, :],
        dimension_numbers=(((1,), (1,)), ((), ())),
        preferred_element_type=jnp.float32,
    ) + b_ref[:, _NH:]
    o_ref[:, _NH:] = h1
    bad = bad0 + jnp.sum(h1 * 0.0, axis=1, keepdims=True)

    @pl.when(jnp.logical_not(jnp.all(bad == 0.0)))
    def _fix_rows():
        o_ref[...] = jnp.where(bad == 0.0, o_ref[...], 0.0)


def kernel(x, W, b):
    M, K = x.shape
    N = W.shape[0]
    b2 = b.reshape(1, N)
    return pl.pallas_call(
        _mm_filter_kernel,
        grid=(M // _TM,),
        in_specs=[
            pl.BlockSpec((_TM, K), lambda i: (i, 0)),
            pl.BlockSpec((N, K), lambda i: (0, 0)),
            pl.BlockSpec((1, N), lambda i: (0, 0)),
        ],
        out_specs=pl.BlockSpec((_TM, N), lambda i: (i, 0)),
        out_shape=jax.ShapeDtypeStruct((M, N), jnp.float32),
        compiler_params=pltpu.CompilerParams(
            dimension_semantics=("parallel",),
        ),
    )(x, W, b2)
